# Initial kernel scaffold; baseline (speedup 1.0000x reference)
import jax
import jax.numpy as jnp
from jax.experimental import pallas as pl


def kernel(x, q_t, w, b1, alpha, lamda, rho, mu, lin_W, lin_b):
    raise NotImplementedError("write your pallas kernel here")


# speedup vs baseline: 11.2756x; 11.2756x over previous
import jax
import jax.numpy as jnp
from jax import lax
from jax.experimental import pallas as pl
from jax.experimental.pallas import tpu as pltpu

_N = 471
_TOPK = 50
_ROWS = 4096
_BLK = 512
_NBLK = _ROWS // _BLK
_QR = 64
_NS_ITERS = 16
_ROUNDS = 10


def _body(x_ref, qt_ref, w_ref, linw_ref, linb_ref, scal_ref, out_ref, acc_ref):
    i = pl.program_id(0)

    @pl.when(i == 0)
    def _init():
        acc_ref[...] = jnp.zeros_like(acc_ref)

    acc_ref[...] += jnp.sum(x_ref[...], axis=0, keepdims=True)

    @pl.when(i == _NBLK - 1)
    def _admm():
        alpha = scal_ref[0, 0]
        lamda = scal_ref[0, 1]
        rho = scal_ref[0, 2]
        mu = scal_ref[0, 3]
        w = w_ref[...]
        qt = qt_ref[...]
        a_mean = acc_ref[...] * (1.0 / _ROWS)
        w2 = lax.dot_general(a_mean, linw_ref[...], (((1,), (1,)), ((), ())),
                             preferred_element_type=jnp.float32) + linb_ref[...]
        g = lax.dot_general(qt, qt, (((1,), (1,)), ((), ())),
                            preferred_element_type=jnp.float32)
        r = jnp.max(jnp.sum(jnp.abs(g), axis=1))
        xinv = g * (1.0 / (r * r))

        def _ns(_, xk):
            gx = jnp.dot(g, xk, preferred_element_type=jnp.float32)
            return 2.0 * xk - jnp.dot(xk, gx, preferred_element_type=jnp.float32)

        xinv = lax.fori_loop(0, _NS_ITERS, _ns, xinv)
        p = alpha * jnp.dot(xinv, qt, preferred_element_type=jnp.float32)
        ones = jnp.ones_like(w)

        def _round(_, carry):
            z, u = carry
            v = u - rho * (z - w)
            theta = lax.dot_general(v, p, (((1,), (1,)), ((), ())),
                                    preferred_element_type=jnp.float32)
            b = w + (1.0 / _N) * jnp.dot(theta, qt,
                                         preferred_element_type=jnp.float32)
            grad = (w2 + rho * (z - b) + u
                    + (2.0 * lamda) * (jnp.sum(z) - 1.0) * ones
                    + (2.0 * lamda) * jnp.minimum(0.0, z))
            z2 = jnp.maximum(z - mu * grad, 0.0)
            zc = z2.reshape(_N, 1)
            rank = jnp.sum((zc > z2).astype(jnp.float32), axis=0, keepdims=True)
            z3 = jnp.where(rank < float(_TOPK), z2, 0.0)
            u2 = u + rho * (z3 - b)
            return z3, u2

        z0 = jnp.zeros_like(w)
        z, _u = lax.fori_loop(0, _ROUNDS, _round, (z0, z0))
        out_ref[...] = z / (jnp.sum(z) + 1e-8)


def kernel(x, q_t, w, b1, alpha, lamda, rho, mu, lin_W, lin_b):
    del b1
    w2d = w.reshape(1, _N).astype(jnp.float32)
    linb2d = lin_b.reshape(1, _N).astype(jnp.float32)
    scal = jnp.concatenate([alpha, lamda, rho, mu]).reshape(1, 4).astype(jnp.float32)
    out = pl.pallas_call(
        _body,
        grid=(_NBLK,),
        in_specs=[
            pl.BlockSpec((_BLK, _N), lambda i: (i, 0)),
            pl.BlockSpec((_QR, _N), lambda i: (0, 0)),
            pl.BlockSpec((1, _N), lambda i: (0, 0)),
            pl.BlockSpec((_N, _N), lambda i: (0, 0)),
            pl.BlockSpec((1, _N), lambda i: (0, 0)),
            pl.BlockSpec((1, 4), lambda i: (0, 0)),
        ],
        out_specs=pl.BlockSpec((1, _N), lambda i: (0, 0)),
        out_shape=jax.ShapeDtypeStruct((1, _N), jnp.float32),
        scratch_shapes=[pltpu.VMEM((1, _N), jnp.float32)],
    )(x, q_t, w2d, lin_W, linb2d, scal)
    return out.reshape(_N)


# speedup vs baseline: 13.0728x; 1.1594x over previous
import jax
import jax.numpy as jnp
from jax import lax
from jax.experimental import pallas as pl
from jax.experimental.pallas import tpu as pltpu

_N = 471
_TOPK = 50
_ROWS = 4096
_BLK = 1024
_NBLK = _ROWS // _BLK
_QR = 64
_NS_ITERS = 12
_ROUNDS = 10


def _body(x_ref, qt_ref, w_ref, linw_ref, linb_ref, scal_ref, out_ref, acc_ref):
    i = pl.program_id(0)

    @pl.when(i == 0)
    def _init():
        acc_ref[...] = jnp.zeros_like(acc_ref)

    acc_ref[...] += jnp.sum(x_ref[...], axis=0, keepdims=True)

    @pl.when(i == _NBLK - 1)
    def _admm():
        alpha = scal_ref[0, 0]
        lamda = scal_ref[0, 1]
        rho = scal_ref[0, 2]
        mu = scal_ref[0, 3]
        w = w_ref[...]
        qt = qt_ref[...]
        a_mean = acc_ref[...] * (1.0 / _ROWS)
        w2 = lax.dot_general(a_mean, linw_ref[...], (((1,), (1,)), ((), ())),
                             preferred_element_type=jnp.float32) + linb_ref[...]
        g = lax.dot_general(qt, qt, (((1,), (1,)), ((), ())),
                            preferred_element_type=jnp.float32)
        r = jnp.max(jnp.sum(jnp.abs(g), axis=1))
        xinv = g * (1.0 / (r * r))
        for _ in range(_NS_ITERS):
            gx = jnp.dot(g, xinv, preferred_element_type=jnp.float32)
            xinv = 2.0 * xinv - jnp.dot(xinv, gx,
                                        preferred_element_type=jnp.float32)
        t1 = jnp.dot(xinv, qt, preferred_element_type=jnp.float32)
        m = (alpha / _N) * lax.dot_general(
            qt, t1, (((0,), (0,)), ((), ())),
            preferred_element_type=jnp.float32)
        ones = jnp.ones_like(w)

        z = jnp.zeros_like(w)
        v = rho * w
        for _ in range(_ROUNDS):
            b = w + lax.dot_general(v, m, (((1,), (0,)), ((), ())),
                                    preferred_element_type=jnp.float32)
            u = v + rho * (z - w)
            grad = (w2 + rho * (z - b) + u
                    + (2.0 * lamda) * (jnp.sum(z) - 1.0) * ones
                    + (2.0 * lamda) * jnp.minimum(0.0, z))
            z2 = jnp.maximum(z - mu * grad, 0.0)
            zc = z2.reshape(_N, 1)
            rank = jnp.sum((zc > z2).astype(jnp.float32), axis=0, keepdims=True)
            z = jnp.where(rank < float(_TOPK), z2, 0.0)
            v = v + rho * (z - b)
        out_ref[...] = z / (jnp.sum(z) + 1e-8)


def kernel(x, q_t, w, b1, alpha, lamda, rho, mu, lin_W, lin_b):
    del b1
    w2d = w.reshape(1, _N).astype(jnp.float32)
    linb2d = lin_b.reshape(1, _N).astype(jnp.float32)
    scal = jnp.concatenate([alpha, lamda, rho, mu]).reshape(1, 4).astype(jnp.float32)
    out = pl.pallas_call(
        _body,
        grid=(_NBLK,),
        in_specs=[
            pl.BlockSpec((_BLK, _N), lambda i: (i, 0)),
            pl.BlockSpec((_QR, _N), lambda i: (0, 0)),
            pl.BlockSpec((1, _N), lambda i: (0, 0)),
            pl.BlockSpec((_N, _N), lambda i: (0, 0)),
            pl.BlockSpec((1, _N), lambda i: (0, 0)),
            pl.BlockSpec((1, 4), lambda i: (0, 0)),
        ],
        out_specs=pl.BlockSpec((1, _N), lambda i: (0, 0)),
        out_shape=jax.ShapeDtypeStruct((1, _N), jnp.float32),
        scratch_shapes=[pltpu.VMEM((1, _N), jnp.float32)],
    )(x, q_t, w2d, lin_W, linb2d, scal)
    return out.reshape(_N)
